# Initial kernel scaffold; baseline (speedup 1.0000x reference)
import jax
import jax.numpy as jnp
from jax.experimental import pallas as pl


def kernel(input, target, alpha, one_hot_codes):
    raise NotImplementedError("write your pallas kernel here")


# speedup vs baseline: 28.9602x; 28.9602x over previous
import functools

import jax
import jax.numpy as jnp
from jax import lax
from jax.experimental import pallas as pl
from jax.experimental.pallas import tpu as pltpu
from jax.experimental.pallas import tpu_sc as plsc

C = 21
N = 8
H = 512
W = 512
NC = 2
NS = 16
NW = NC * NS
ROWS_PER_TILE = (N * H) // NW
K = 4
SLABS = ROWS_PER_TILE // K
PIX = K * W
VECS = PIX // 16

_LN2 = 0.6931471805599453
_SQRT2 = 1.4142135623730951


def _log_f32(p):
    bits = plsc.bitcast(p, jnp.int32)
    e = (bits >> 23) - 127
    m = plsc.bitcast((bits & 0x007FFFFF) | 0x3F800000, jnp.float32)
    big = m > _SQRT2
    m = jnp.where(big, m * 0.5, m)
    ef = jnp.where(big, e + 1, e).astype(jnp.float32)
    r = (m - 1.0) / (m + 1.0)
    r2 = r * r
    poly = r * (2.0 + r2 * (0.6666666666666666 + r2 * (0.4 + r2 * (2.0 / 7.0))))
    return ef * _LN2 + poly


def _body(inp, tgt, alf, out, slab_v, tgt_v, alf_v, acc_v):
    c = lax.axis_index("c")
    s = lax.axis_index("s")
    wid = s * NC + c
    n = wid // 4
    h_base = (wid % 4) * ROWS_PER_TILE

    pltpu.sync_copy(alf, alf_v)
    lane = lax.iota(jnp.int32, 16)

    def slab_loop(si, acc):
        h0 = h_base + si * K
        pltpu.sync_copy(inp.at[n, :, pl.ds(h0 * W, PIX)], slab_v)
        pltpu.sync_copy(tgt.at[pl.ds((n * H + h0) * W, PIX)], tgt_v)

        def vec_loop(j, a_in):
            base = j * 16
            t = tgt_v[pl.ds(base, 16)]
            p = plsc.load_gather(slab_v, [t, base + lane]) + 1e-10
            a = plsc.load_gather(alf_v, [t])
            omp = 1.0 - p
            return a_in - a * omp * omp * _log_f32(p)

        return lax.fori_loop(0, VECS, vec_loop, acc)

    acc = lax.fori_loop(0, SLABS, slab_loop, jnp.zeros((16,), jnp.float32))
    acc_v[...] = acc
    pltpu.sync_copy(acc_v, out.at[wid])


@jax.jit
def _focal_partials(inp3, tgt1, alf1):
    mesh = plsc.VectorSubcoreMesh(core_axis_name="c", subcore_axis_name="s")
    return pl.kernel(
        _body,
        out_type=jax.ShapeDtypeStruct((NW, 16), jnp.float32),
        mesh=mesh,
        compiler_params=pltpu.CompilerParams(
            use_tc_tiling_on_sc=False, needs_layout_passes=False
        ),
        scratch_types=[
            pltpu.VMEM((C, PIX), jnp.float32),
            pltpu.VMEM((PIX,), jnp.int32),
            pltpu.VMEM((C,), jnp.float32),
            pltpu.VMEM((16,), jnp.float32),
        ],
    )(inp3, tgt1, alf1)


def kernel(input, target, alpha, one_hot_codes):
    inp3 = input.reshape(N, C, H * W)
    tgt1 = target.reshape(-1).astype(jnp.int32)
    alf1 = alpha.reshape(-1)
    partials = _focal_partials(inp3, tgt1, alf1)
    return jnp.sum(partials) / (N * H * W)


# speedup vs baseline: 34.5528x; 1.1931x over previous
import functools

import jax
import jax.numpy as jnp
from jax import lax
from jax.experimental import pallas as pl
from jax.experimental.pallas import tpu as pltpu
from jax.experimental.pallas import tpu_sc as plsc

C = 21
N = 8
H = 512
W = 512
NC = 2
NS = 16
NW = NC * NS
ROWS_PER_TILE = (N * H) // NW
K = 4
SLABS = ROWS_PER_TILE // K
PIX = K * W
VECS = PIX // 16

_LN2 = 0.6931471805599453
_SQRT2 = 1.4142135623730951


def _log_f32(p):
    bits = plsc.bitcast(p, jnp.int32)
    e = (bits >> 23) - 127
    m = plsc.bitcast((bits & 0x007FFFFF) | 0x3F800000, jnp.float32)
    big = m > _SQRT2
    m = jnp.where(big, m * 0.5, m)
    ef = jnp.where(big, e + 1, e).astype(jnp.float32)
    r = (m - 1.0) / (m + 1.0)
    r2 = r * r
    poly = r * (2.0 + r2 * (0.6666666666666666 + r2 * (0.4 + r2 * (2.0 / 7.0))))
    return ef * _LN2 + poly


def _body(inp, tgt, alf, out, slab_v, tgt_v, alf_v, acc_v, slab_sem, tgt_sem):
    c = lax.axis_index("c")
    s = lax.axis_index("s")
    wid = s * NC + c
    n = wid // 4
    h_base = (wid % 4) * ROWS_PER_TILE

    pltpu.sync_copy(alf, alf_v)
    lane = lax.iota(jnp.int32, 16)

    def start(si, b):
        h0 = h_base + si * K
        pltpu.async_copy(
            inp.at[n, :, pl.ds(h0 * W, PIX)], slab_v.at[b], slab_sem.at[b]
        )
        pltpu.async_copy(
            tgt.at[pl.ds((n * H + h0) * W, PIX)], tgt_v.at[b], tgt_sem.at[b]
        )

    start(0, 0)
    start(1, 1)

    def pair_loop(g, acc):
        for b in range(2):
            si = g * 2 + b
            pltpu.make_async_copy(
                inp.at[n, :, pl.ds(0, PIX)], slab_v.at[b], slab_sem.at[b]
            ).wait()
            pltpu.make_async_copy(
                tgt.at[pl.ds(0, PIX)], tgt_v.at[b], tgt_sem.at[b]
            ).wait()

            def vec_loop(j, a_in, b=b):
                base = j * 16
                t = tgt_v[b, pl.ds(base, 16)]
                p = plsc.load_gather(slab_v.at[b], [t, base + lane]) + 1e-10
                a = plsc.load_gather(alf_v, [t])
                omp = 1.0 - p
                return a_in - a * omp * omp * _log_f32(p)

            acc = lax.fori_loop(0, VECS, vec_loop, acc)

            @pl.when(si + 2 < SLABS)
            def _():
                start(si + 2, b)

        return acc

    acc = lax.fori_loop(0, SLABS // 2, pair_loop, jnp.zeros((16,), jnp.float32))
    acc_v[...] = acc
    pltpu.sync_copy(acc_v, out.at[wid])


@jax.jit
def _focal_partials(inp3, tgt1, alf1):
    mesh = plsc.VectorSubcoreMesh(core_axis_name="c", subcore_axis_name="s")
    return pl.kernel(
        _body,
        out_type=jax.ShapeDtypeStruct((NW, 16), jnp.float32),
        mesh=mesh,
        compiler_params=pltpu.CompilerParams(
            use_tc_tiling_on_sc=False, needs_layout_passes=False
        ),
        scratch_types=[
            pltpu.VMEM((2, C, PIX), jnp.float32),
            pltpu.VMEM((2, PIX), jnp.int32),
            pltpu.VMEM((C,), jnp.float32),
            pltpu.VMEM((16,), jnp.float32),
            pltpu.SemaphoreType.DMA((2,)),
            pltpu.SemaphoreType.DMA((2,)),
        ],
    )(inp3, tgt1, alf1)


def kernel(input, target, alpha, one_hot_codes):
    inp3 = input.reshape(N, C, H * W)
    tgt1 = target.reshape(-1).astype(jnp.int32)
    alf1 = alpha.reshape(-1)
    partials = _focal_partials(inp3, tgt1, alf1)
    return jnp.sum(partials) / (N * H * W)
